# baseline (speedup 1.0000x reference)
import functools

import jax
import jax.numpy as jnp
from jax.experimental import pallas as pl
from jax.experimental.pallas import tpu as pltpu

TOPK = 8
NEXP = 64
HID = 4096
BLOCK_T = 512


def _gate_block(x_ref, wt_ref, w_out_ref, i_out_ref):
    x = x_ref[...]
    wt = wt_ref[...]
    logits = jnp.dot(x, wt, preferred_element_type=jnp.float32,
                     precision=jax.lax.Precision.DEFAULT)
    lt = logits.T
    cmax = jnp.max(lt, axis=0, keepdims=True)
    denom = jnp.sum(jnp.exp(lt - cmax), axis=0, keepdims=True)
    iota = jax.lax.broadcasted_iota(jnp.int32, lt.shape, 0)
    work = lt
    vals, idxs = [], []
    for _ in range(TOPK):
        m = jnp.max(work, axis=0, keepdims=True)
        cand = jnp.where(work >= m, iota, NEXP)
        idx = jnp.min(cand, axis=0, keepdims=True)
        vals.append(m)
        idxs.append(idx)
        work = jnp.where(iota == idx, -jnp.inf, work)
    valcat = jnp.concatenate(vals, axis=0)
    idxcat = jnp.concatenate(idxs, axis=0)
    w_out_ref[...] = (jnp.exp(valcat - cmax) / denom).T
    i_out_ref[...] = idxcat.T


@jax.jit
def kernel(hidden_states, weight):
    h = hidden_states.shape[-1]
    x = hidden_states.reshape(-1, h).astype(jnp.float32)
    n_tok = x.shape[0]
    wt = weight.astype(jnp.float32).T
    grid = (n_tok // BLOCK_T,)
    w_out, i_out = pl.pallas_call(
        _gate_block,
        grid=grid,
        in_specs=[
            pl.BlockSpec((BLOCK_T, HID), lambda i: (i, 0)),
            pl.BlockSpec((HID, NEXP), lambda i: (0, 0)),
        ],
        out_specs=[
            pl.BlockSpec((BLOCK_T, TOPK), lambda i: (i, 0)),
            pl.BlockSpec((BLOCK_T, TOPK), lambda i: (i, 0)),
        ],
        out_shape=[
            jax.ShapeDtypeStruct((n_tok, TOPK), jnp.float32),
            jax.ShapeDtypeStruct((n_tok, TOPK), jnp.int32),
        ],
        compiler_params=pltpu.CompilerParams(
            dimension_semantics=("parallel",),
        ),
    )(x, wt)
    return w_out, i_out


# speedup vs baseline: 1.0013x; 1.0013x over previous
import jax
import jax.numpy as jnp
from jax.experimental import pallas as pl
from jax.experimental.pallas import tpu as pltpu

TOPK = 8
NEXP = 64
HID = 4096
BLOCK_T = 512
N_TOK = 8192
GRID = N_TOK // (2 * BLOCK_T)


def _topk_softmax(logits):
    lt = logits.T
    cmax = jnp.max(lt, axis=0, keepdims=True)
    denom = jnp.sum(jnp.exp(lt - cmax), axis=0, keepdims=True)
    iota = jax.lax.broadcasted_iota(jnp.int32, lt.shape, 0)
    work = lt
    vals, idxs = [], []
    for _ in range(TOPK):
        m = jnp.max(work, axis=0, keepdims=True)
        cand = jnp.where(work >= m, iota, NEXP)
        idx = jnp.min(cand, axis=0, keepdims=True)
        vals.append(m)
        idxs.append(idx)
        work = jnp.where(iota == idx, -jnp.inf, work)
    valcat = jnp.concatenate(vals, axis=0)
    idxcat = jnp.concatenate(idxs, axis=0)
    w = (jnp.exp(valcat - cmax) / denom).T
    return w, idxcat.T


def _gate_block(xa_ref, xb_ref, wt_ref, wa_ref, ia_ref, wb_ref, ib_ref):
    wt = wt_ref[...]
    la = jnp.dot(xa_ref[0], wt, preferred_element_type=jnp.float32,
                 precision=jax.lax.Precision.DEFAULT)
    lb = jnp.dot(xb_ref[0], wt, preferred_element_type=jnp.float32,
                 precision=jax.lax.Precision.DEFAULT)
    wa, ia = _topk_softmax(la)
    wa_ref[0] = wa
    ia_ref[0] = ia
    wb, ib = _topk_softmax(lb)
    wb_ref[0] = wb
    ib_ref[0] = ib


@jax.jit
def kernel(hidden_states, weight):
    h = hidden_states.shape[-1]
    x = hidden_states.reshape(2 * GRID, BLOCK_T, h).astype(jnp.float32)
    wt = weight.astype(jnp.float32).T
    half = jax.ShapeDtypeStruct((GRID, BLOCK_T, TOPK), jnp.float32)
    half_i = jax.ShapeDtypeStruct((GRID, BLOCK_T, TOPK), jnp.int32)
    wa, ia, wb, ib = pl.pallas_call(
        _gate_block,
        grid=(GRID,),
        in_specs=[
            pl.BlockSpec((1, BLOCK_T, HID), lambda i: (i, 0, 0)),
            pl.BlockSpec((1, BLOCK_T, HID), lambda i: (i + GRID, 0, 0)),
            pl.BlockSpec((HID, NEXP), lambda i: (0, 0)),
        ],
        out_specs=[
            pl.BlockSpec((1, BLOCK_T, TOPK), lambda i: (i, 0, 0)),
            pl.BlockSpec((1, BLOCK_T, TOPK), lambda i: (i, 0, 0)),
            pl.BlockSpec((1, BLOCK_T, TOPK), lambda i: (i, 0, 0)),
            pl.BlockSpec((1, BLOCK_T, TOPK), lambda i: (i, 0, 0)),
        ],
        out_shape=[half, half_i, half, half_i],
        compiler_params=pltpu.CompilerParams(
            dimension_semantics=("arbitrary",),
        ),
    )(x, x, wt)
    w_out = jnp.concatenate([wa, wb], axis=0).reshape(N_TOK, TOPK)
    i_out = jnp.concatenate([ia, ib], axis=0).reshape(N_TOK, TOPK)
    return w_out, i_out


# speedup vs baseline: 1.0273x; 1.0259x over previous
import functools

import jax
import jax.numpy as jnp
from jax.experimental import pallas as pl
from jax.experimental.pallas import tpu as pltpu

TOPK = 8
NEXP = 64
HID = 4096
BLOCK_T = 1024


def _gate_block(x_ref, wt_ref, w_out_ref, i_out_ref):
    x = x_ref[...]
    wt = wt_ref[...]
    logits = jnp.dot(x, wt, preferred_element_type=jnp.float32,
                     precision=jax.lax.Precision.DEFAULT)
    lt = logits.T
    cmax = jnp.max(lt, axis=0, keepdims=True)
    denom = jnp.sum(jnp.exp(lt - cmax), axis=0, keepdims=True)
    iota = jax.lax.broadcasted_iota(jnp.int32, lt.shape, 0)
    work = lt
    vals, idxs = [], []
    for _ in range(TOPK):
        m = jnp.max(work, axis=0, keepdims=True)
        cand = jnp.where(work >= m, iota, NEXP)
        idx = jnp.min(cand, axis=0, keepdims=True)
        vals.append(m)
        idxs.append(idx)
        work = jnp.where(iota == idx, -jnp.inf, work)
    valcat = jnp.concatenate(vals, axis=0)
    idxcat = jnp.concatenate(idxs, axis=0)
    w_out_ref[...] = (jnp.exp(valcat - cmax) / denom).T
    i_out_ref[...] = idxcat.T


@jax.jit
def kernel(hidden_states, weight):
    h = hidden_states.shape[-1]
    x = hidden_states.reshape(-1, h).astype(jnp.float32)
    n_tok = x.shape[0]
    wt = weight.astype(jnp.float32).T
    grid = (n_tok // BLOCK_T,)
    w_out, i_out = pl.pallas_call(
        _gate_block,
        grid=grid,
        in_specs=[
            pl.BlockSpec((BLOCK_T, HID), lambda i: (i, 0)),
            pl.BlockSpec((HID, NEXP), lambda i: (0, 0)),
        ],
        out_specs=[
            pl.BlockSpec((BLOCK_T, TOPK), lambda i: (i, 0)),
            pl.BlockSpec((BLOCK_T, TOPK), lambda i: (i, 0)),
        ],
        out_shape=[
            jax.ShapeDtypeStruct((n_tok, TOPK), jnp.float32),
            jax.ShapeDtypeStruct((n_tok, TOPK), jnp.int32),
        ],
        compiler_params=pltpu.CompilerParams(
            dimension_semantics=("arbitrary",),
        ),
    )(x, wt)
    return w_out, i_out


# speedup vs baseline: 1.0931x; 1.0641x over previous
import functools

import jax
import jax.numpy as jnp
from jax.experimental import pallas as pl
from jax.experimental.pallas import tpu as pltpu

TOPK = 8
NEXP = 64
HID = 4096
BLOCK_T = 1024


def _gate_block(x_ref, w_ref, w_out_ref, i_out_ref):
    x = x_ref[...]
    w = w_ref[...]
    logits = jax.lax.dot_general(
        x, w, (((1,), (1,)), ((), ())),
        preferred_element_type=jnp.float32,
        precision=jax.lax.Precision.DEFAULT)
    lt = logits.T
    cmax = jnp.max(lt, axis=0, keepdims=True)
    denom = jnp.sum(jnp.exp(lt - cmax), axis=0, keepdims=True)
    iota = jax.lax.broadcasted_iota(jnp.int32, lt.shape, 0)
    work = lt
    vals, idxs = [], []
    for _ in range(TOPK):
        m = jnp.max(work, axis=0, keepdims=True)
        cand = jnp.where(work >= m, iota, NEXP)
        idx = jnp.min(cand, axis=0, keepdims=True)
        vals.append(m)
        idxs.append(idx)
        work = jnp.where(iota == idx, -jnp.inf, work)
    valcat = jnp.concatenate(vals, axis=0)
    idxcat = jnp.concatenate(idxs, axis=0)
    w_out_ref[...] = (jnp.exp(valcat - cmax) / denom).T
    i_out_ref[...] = idxcat.T


@jax.jit
def kernel(hidden_states, weight):
    h = hidden_states.shape[-1]
    x = hidden_states.reshape(-1, h).astype(jnp.float32)
    n_tok = x.shape[0]
    w = weight.astype(jnp.float32)
    grid = (n_tok // BLOCK_T,)
    w_out, i_out = pl.pallas_call(
        _gate_block,
        grid=grid,
        in_specs=[
            pl.BlockSpec((BLOCK_T, HID), lambda i: (i, 0)),
            pl.BlockSpec((NEXP, HID), lambda i: (0, 0)),
        ],
        out_specs=[
            pl.BlockSpec((BLOCK_T, TOPK), lambda i: (i, 0)),
            pl.BlockSpec((BLOCK_T, TOPK), lambda i: (i, 0)),
        ],
        out_shape=[
            jax.ShapeDtypeStruct((n_tok, TOPK), jnp.float32),
            jax.ShapeDtypeStruct((n_tok, TOPK), jnp.int32),
        ],
        compiler_params=pltpu.CompilerParams(
            dimension_semantics=("arbitrary",),
        ),
    )(x, w)
    return w_out, i_out
